# baseline (speedup 1.0000x reference)
import functools

import jax
import jax.numpy as jnp
from jax import lax
from jax.experimental import pallas as pl
from jax.experimental.pallas import tpu as pltpu
from jax.experimental.pallas import tpu_sc as plsc

N = 10000
NP = 10240
E = 160000
EP = 172032
C = 2048
NB = 1024
F32 = jnp.float32

_mesh = plsc.VectorSubcoreMesh(core_axis_name="c", subcore_axis_name="s")
_CP_SC = pltpu.CompilerParams(needs_layout_passes=False)


def _tc_pre_body(x_ref, w_ref, a_ref, hT_ref, al_ref, gmax_ref):
    hT = lax.dot_general(w_ref[...], x_ref[...], (((0,), (1,)), ((), ())),
                         preferred_element_type=F32)
    hT_ref[...] = hT
    al = lax.dot_general(a_ref[...], hT, (((0,), (0,)), ((), ())),
                         preferred_element_type=F32)
    al_ref[...] = al
    rm = jnp.max(al, axis=1, keepdims=True)
    rmb = lax.broadcast_in_dim(rm, (8, 128), (0, 1))

    @pl.when(pl.program_id(0) == 0)
    def _():
        gmax_ref[...] = rmb

    @pl.when(pl.program_id(0) != 0)
    def _():
        gmax_ref[...] = jnp.maximum(gmax_ref[...], rmb)


def _tc_pre(xp, W, A, dh):
    din = xp.shape[1]
    return pl.pallas_call(
        _tc_pre_body,
        grid=(NP // NB,),
        in_specs=[
            pl.BlockSpec((NB, din), lambda i: (i, 0)),
            pl.BlockSpec((din, dh), lambda i: (0, 0)),
            pl.BlockSpec((dh, 8), lambda i: (0, 0)),
        ],
        out_specs=[
            pl.BlockSpec((dh, NB), lambda i: (0, i)),
            pl.BlockSpec((8, NB), lambda i: (0, i)),
            pl.BlockSpec((8, 128), lambda i: (0, 0)),
        ],
        out_shape=[
            jax.ShapeDtypeStruct((dh, NP), F32),
            jax.ShapeDtypeStruct((8, NP), F32),
            jax.ShapeDtypeStruct((8, 128), F32),
        ],
    )(xp, W, A)


def _tc_mid_body(acc_ref, den_ref, b_ref, w_ref, a_ref,
                 zT_ref, al_ref, gmax_ref):
    i = pl.program_id(0)
    acc = acc_ref[...]
    den = den_ref[...]
    col = lax.broadcasted_iota(jnp.int32, (1, NB), 1) + i * NB
    valid = col < N
    acc = jnp.where(lax.broadcast_in_dim(valid, (256, NB), (0, 1)), acc, 0.0)
    den = jnp.where(lax.broadcast_in_dim(valid, (4, NB), (0, 1)), den, 1.0)
    acc3 = acc.reshape(4, 64, NB)
    den3 = lax.broadcast_in_dim(den, (4, 64, NB), (0, 2))
    h = acc3 / (den3 + 1e-16) + b_ref[...].reshape(4, 64, 1)
    h = h.reshape(256, NB)
    h = jnp.where(h > 0, h, jnp.exp(h) - 1.0)
    z = lax.dot_general(w_ref[...], h, (((0,), (0,)), ((), ())),
                        preferred_element_type=F32)
    zT_ref[...] = z
    al2 = lax.dot_general(a_ref[...], z, (((0,), (0,)), ((), ())),
                          preferred_element_type=F32)
    al2p = jnp.concatenate([al2, jnp.full((6, NB), -1e30, F32)], axis=0)
    al_ref[...] = al2p
    rm = jnp.max(al2p, axis=1, keepdims=True)
    rmb = lax.broadcast_in_dim(rm, (8, 128), (0, 1))

    @pl.when(i == 0)
    def _():
        gmax_ref[...] = rmb

    @pl.when(i != 0)
    def _():
        gmax_ref[...] = jnp.maximum(gmax_ref[...], rmb)


def _tc_mid(accT, denT, b1c, W2, A2):
    return pl.pallas_call(
        _tc_mid_body,
        grid=(NP // NB,),
        in_specs=[
            pl.BlockSpec((256, NB), lambda i: (0, i)),
            pl.BlockSpec((4, NB), lambda i: (0, i)),
            pl.BlockSpec((256, 1), lambda i: (0, 0)),
            pl.BlockSpec((256, 128), lambda i: (0, 0)),
            pl.BlockSpec((128, 2), lambda i: (0, 0)),
        ],
        out_specs=[
            pl.BlockSpec((128, NB), lambda i: (0, i)),
            pl.BlockSpec((8, NB), lambda i: (0, i)),
            pl.BlockSpec((8, 128), lambda i: (0, 0)),
        ],
        out_shape=[
            jax.ShapeDtypeStruct((128, NP), F32),
            jax.ShapeDtypeStruct((8, NP), F32),
            jax.ShapeDtypeStruct((8, 128), F32),
        ],
    )(accT, denT, b1c, W2, A2)


def _tc_post_body(acc_ref, den_ref, b_ref, eye_ref, out_ref):
    acc = acc_ref[...]
    den = den_ref[...]
    h = acc / (lax.broadcast_in_dim(den, (128, NB), (0, 1)) + 1e-16)
    h = h + b_ref[...]
    h = jnp.where(h > 0, h, jnp.exp(h) - 1.0)
    out_ref[...] = lax.dot_general(h, eye_ref[...], (((0,), (0,)), ((), ())),
                                   preferred_element_type=F32)


def _tc_post(acc2T, den2, b2c, eye):
    return pl.pallas_call(
        _tc_post_body,
        grid=(NP // NB,),
        in_specs=[
            pl.BlockSpec((128, NB), lambda i: (0, i)),
            pl.BlockSpec((1, NB), lambda i: (0, i)),
            pl.BlockSpec((128, 1), lambda i: (0, 0)),
            pl.BlockSpec((128, 128), lambda i: (0, 0)),
        ],
        out_specs=pl.BlockSpec((NB, 128), lambda i: (i, 0)),
        out_shape=jax.ShapeDtypeStruct((NP, 128), F32),
    )(acc2T, den2, b2c, eye)


def _make_edge_kernel(heads, featc):
    nunits = featc // 4
    units_per_tec = nunits // 32
    dst_row = 4 if heads == 4 else 1
    chunks = EP // C
    if heads == 4:
        pha_range, pha_ca = EP // 4, 2048
    else:
        pha_range, pha_ca = EP // 16, 1344
    pha_chunks = pha_range // pha_ca

    @functools.partial(
        pl.kernel,
        out_type=(jax.ShapeDtypeStruct((featc * NP,), F32),
                  jax.ShapeDtypeStruct((8 * NP,), F32),
                  jax.ShapeDtypeStruct((heads * EP,), F32)),
        mesh=_mesh,
        compiler_params=_CP_SC,
        scratch_types=(
            [pltpu.VMEM((NP,), F32) for _ in range(4)]
            + [pltpu.VMEM((NP,), F32) for _ in range(4)]
            + [
                pltpu.VMEM((NP,), F32),
                pltpu.VMEM((NP,), F32),
                pltpu.VMEM((NP,), F32),
                pltpu.VMEM((C,), jnp.int32),
                pltpu.VMEM((C,), jnp.int32),
                pltpu.VMEM((C,), jnp.int32),
                pltpu.VMEM((C,), jnp.int32),
                pltpu.VMEM((C,), F32),
                pltpu.VMEM((C,), F32),
                pltpu.VMEM((128,), F32),
                pltpu.VMEM((128,), F32),
                pltpu.SemaphoreType.DMA,
                pltpu.SemaphoreType.DMA,
                pltpu.SemaphoreType.DMA,
                pltpu.SemaphoreType.DMA,
                pltpu.SemaphoreType.DMA,
                pltpu.SemaphoreType.DMA,
            ]
        ),
    )
    def edge_kernel(tabT, alphaT, gmaxrow, src, dst, accT_o, den_o, exS,
                    t0, t1, t2, t3, a0, a1, a2, a3,
                    asr, ads, accd, sv0, dv0, sv1, dv1, eb0, eb1, gm1, gm2,
                    ss0, sd0, ss1, sd1, se0, se1):
        tabs = (t0, t1, t2, t3)
        accs = (a0, a1, a2, a3)
        cid = lax.axis_index("c")
        sid = lax.axis_index("s")
        wid = sid * 2 + cid
        zeros = jnp.zeros((16,), F32)

        if heads == 4:
            a_head = sid // 4
            a_sub = sid % 4
        else:
            a_head = sid * 0
            a_sub = sid
        a_base = a_head * EP + a_sub * pha_range
        pltpu.sync_copy(alphaT.at[pl.ds(a_head * NP, NP)], asr)
        pltpu.sync_copy(alphaT.at[pl.ds((dst_row + a_head) * NP, NP)], ads)
        pltpu.sync_copy(gmaxrow.at[pl.ds(a_head * 128, 128)], gm1)
        pltpu.sync_copy(gmaxrow.at[pl.ds((dst_row + a_head) * 128, 128)], gm2)
        b = gm1[pl.ds(0, 16)] + gm2[pl.ds(0, 16)]
        g = jnp.maximum(b, 0.2 * b)

        def _pha(i, carry):
            off = a_sub * pha_range + i * pha_ca
            pltpu.sync_copy(src.at[pl.ds(off, pha_ca)], sv0.at[pl.ds(0, pha_ca)])
            pltpu.sync_copy(dst.at[pl.ds(off, pha_ca)], dv0.at[pl.ds(0, pha_ca)])

            @plsc.parallel_loop(0, pha_ca, 16, unroll=4)
            def _body(o):
                s = sv0[pl.ds(o, 16)]
                d = dv0[pl.ds(o, 16)]
                e = plsc.load_gather(asr, [s]) + plsc.load_gather(ads, [d])
                e = jnp.maximum(e, 0.2 * e)
                eb0[pl.ds(o, 16)] = jnp.exp(e - g)

            pltpu.sync_copy(eb0.at[pl.ds(0, pha_ca)],
                            exS.at[pl.ds(a_base + i * pha_ca, pha_ca)])
            return carry

        lax.fori_loop(0, pha_chunks, _pha, 0)
        plsc.subcore_barrier()

        for t in range(units_per_tec):
            u = wid * units_per_tec + t
            head = (u // 16) if heads == 4 else (u * 0)
            for f in range(4):
                pltpu.sync_copy(tabT.at[pl.ds((u * 4 + f) * NP, NP)], tabs[f])

            @plsc.parallel_loop(0, NP, 16, unroll=8)
            def _zero(o):
                for f in range(4):
                    accs[f][pl.ds(o, 16)] = zeros
                accd[pl.ds(o, 16)] = zeros

            def _start(ci, svb, dvb, evb, sems):
                pltpu.async_copy(src.at[pl.ds(ci * C, C)], svb, sems[0])
                pltpu.async_copy(dst.at[pl.ds(ci * C, C)], dvb, sems[1])
                pltpu.async_copy(exS.at[pl.ds(head * EP + ci * C, C)],
                                 evb, sems[2])

            def _wait(svb, dvb, evb, sems):
                pltpu.make_async_copy(src.at[pl.ds(0, C)], svb, sems[0]).wait()
                pltpu.make_async_copy(dst.at[pl.ds(0, C)], dvb, sems[1]).wait()
                pltpu.make_async_copy(exS.at[pl.ds(0, C)], evb, sems[2]).wait()

            def _run(svb, dvb, evb):
                @plsc.parallel_loop(0, C, 16, unroll=4)
                def _body(o):
                    s = svb[pl.ds(o, 16)]
                    d = dvb[pl.ds(o, 16)]
                    ex = evb[pl.ds(o, 16)]
                    for f in range(4):
                        tv = plsc.load_gather(tabs[f], [s])
                        plsc.addupdate_scatter(accs[f], [d], tv * ex)
                    plsc.addupdate_scatter(accd, [d], ex)

            _start(0, sv0, dv0, eb0, (ss0, sd0, se0))

            def _pair(j, carry):
                ci = 2 * j
                _start(ci + 1, sv1, dv1, eb1, (ss1, sd1, se1))
                _wait(sv0, dv0, eb0, (ss0, sd0, se0))
                _run(sv0, dv0, eb0)
                _start(jnp.minimum(ci + 2, chunks - 1), sv0, dv0, eb0,
                       (ss0, sd0, se0))
                _wait(sv1, dv1, eb1, (ss1, sd1, se1))
                _run(sv1, dv1, eb1)
                return carry

            lax.fori_loop(0, chunks // 2, _pair, 0)
            _wait(sv0, dv0, eb0, (ss0, sd0, se0))
            for f in range(4):
                pltpu.sync_copy(accs[f], accT_o.at[pl.ds((u * 4 + f) * NP, NP)])
            is_aug = (u % 16 == 0) if heads == 4 else (u == 0)

            @pl.when(is_aug)
            def _():
                pltpu.sync_copy(accd, den_o.at[pl.ds(head * NP, NP)])

    return edge_kernel


_edge_l1 = _make_edge_kernel(4, 256)
_edge_l2 = _make_edge_kernel(1, 128)


def _branch(x, edge_index, p1, p2):
    W1, as1, ad1, b1 = p1
    W2, as2, ad2, b2 = p2

    loop = jnp.arange(N, dtype=edge_index.dtype)
    src = jnp.concatenate([edge_index[0], loop])
    dst = jnp.concatenate([edge_index[1], loop])
    pad = jnp.full((EP - E - N,), NP - 1, dtype=src.dtype)
    src = jnp.concatenate([src, pad])
    dst = jnp.concatenate([dst, pad])

    xp = jnp.pad(x, ((0, NP - N), (0, 0)))

    eye4 = jnp.eye(4, dtype=F32)
    A1s = jnp.einsum("hc,hk->hck", as1, eye4).reshape(256, 4)
    A1d = jnp.einsum("hc,hk->hck", ad1, eye4).reshape(256, 4)
    A1 = jnp.concatenate([A1s, A1d], axis=1)
    A2 = jnp.stack([as2[0], ad2[0]], axis=1)

    h1T, alphaT, gmaxrow = _tc_pre(xp, W1, A1, 256)
    accT, denT, _ex1 = _edge_l1(h1T.reshape(-1), alphaT.reshape(-1),
                                gmaxrow.reshape(-1), src, dst)
    zT, alphaT2, gmax2row = _tc_mid(accT.reshape(256, NP),
                                    denT.reshape(8, NP)[:4],
                                    b1[:, None], W2, A2)
    acc2T, den2, _ex2 = _edge_l2(zT.reshape(-1), alphaT2.reshape(-1),
                                 gmax2row.reshape(-1), src, dst)
    outp = _tc_post(acc2T.reshape(128, NP), den2.reshape(8, NP)[:1],
                    b2[:, None], jnp.eye(128, dtype=F32))
    return outp[:N]


def kernel(x0, x1, edge_index0, edge_index1, W1_0, as1_0, ad1_0, b1_0, W2_0, as2_0, ad2_0, b2_0, W1_1, as1_1, ad1_1, b1_1, W2_1, as2_1, ad2_1, b2_1):
    out0 = _branch(x0, edge_index0, (W1_0, as1_0, ad1_0, b1_0), (W2_0, as2_0, ad2_0, b2_0))
    out1 = _branch(x1, edge_index1, (W1_1, as1_1, ad1_1, b1_1), (W2_1, as2_1, ad2_1, b2_1))
    return jnp.concatenate([out0, out1], axis=0)


# speedup vs baseline: 1.0216x; 1.0216x over previous
import functools

import jax
import jax.numpy as jnp
from jax import lax
from jax.experimental import pallas as pl
from jax.experimental.pallas import tpu as pltpu
from jax.experimental.pallas import tpu_sc as plsc

N = 10000
NP = 10240
E = 160000
EP = 172032
C = 4096
NB = 1024
F32 = jnp.float32

_mesh = plsc.VectorSubcoreMesh(core_axis_name="c", subcore_axis_name="s")
_CP_SC = pltpu.CompilerParams(needs_layout_passes=False)


def _tc_pre_body(x_ref, w_ref, a_ref, hT_ref, al_ref, gmax_ref):
    hT = lax.dot_general(w_ref[...], x_ref[...], (((0,), (1,)), ((), ())),
                         preferred_element_type=F32)
    hT_ref[...] = hT
    al = lax.dot_general(a_ref[...], hT, (((0,), (0,)), ((), ())),
                         preferred_element_type=F32)
    al_ref[...] = al
    rm = jnp.max(al, axis=1, keepdims=True)
    rmb = lax.broadcast_in_dim(rm, (8, 128), (0, 1))

    @pl.when(pl.program_id(0) == 0)
    def _():
        gmax_ref[...] = rmb

    @pl.when(pl.program_id(0) != 0)
    def _():
        gmax_ref[...] = jnp.maximum(gmax_ref[...], rmb)


def _tc_pre(xp, W, A, dh):
    din = xp.shape[1]
    return pl.pallas_call(
        _tc_pre_body,
        grid=(NP // NB,),
        in_specs=[
            pl.BlockSpec((NB, din), lambda i: (i, 0)),
            pl.BlockSpec((din, dh), lambda i: (0, 0)),
            pl.BlockSpec((dh, 8), lambda i: (0, 0)),
        ],
        out_specs=[
            pl.BlockSpec((dh, NB), lambda i: (0, i)),
            pl.BlockSpec((8, NB), lambda i: (0, i)),
            pl.BlockSpec((8, 128), lambda i: (0, 0)),
        ],
        out_shape=[
            jax.ShapeDtypeStruct((dh, NP), F32),
            jax.ShapeDtypeStruct((8, NP), F32),
            jax.ShapeDtypeStruct((8, 128), F32),
        ],
    )(xp, W, A)


def _tc_mid_body(acc_ref, den_ref, b_ref, w_ref, a_ref,
                 zT_ref, al_ref, gmax_ref):
    i = pl.program_id(0)
    acc = acc_ref[...]
    den = den_ref[...]
    col = lax.broadcasted_iota(jnp.int32, (1, NB), 1) + i * NB
    valid = col < N
    acc = jnp.where(lax.broadcast_in_dim(valid, (256, NB), (0, 1)), acc, 0.0)
    den = jnp.where(lax.broadcast_in_dim(valid, (4, NB), (0, 1)), den, 1.0)
    acc3 = acc.reshape(4, 64, NB)
    den3 = lax.broadcast_in_dim(den, (4, 64, NB), (0, 2))
    h = acc3 / (den3 + 1e-16) + b_ref[...].reshape(4, 64, 1)
    h = h.reshape(256, NB)
    h = jnp.where(h > 0, h, jnp.exp(h) - 1.0)
    z = lax.dot_general(w_ref[...], h, (((0,), (0,)), ((), ())),
                        preferred_element_type=F32)
    zT_ref[...] = z
    al2 = lax.dot_general(a_ref[...], z, (((0,), (0,)), ((), ())),
                          preferred_element_type=F32)
    al2p = jnp.concatenate([al2, jnp.full((6, NB), -1e30, F32)], axis=0)
    al_ref[...] = al2p
    rm = jnp.max(al2p, axis=1, keepdims=True)
    rmb = lax.broadcast_in_dim(rm, (8, 128), (0, 1))

    @pl.when(i == 0)
    def _():
        gmax_ref[...] = rmb

    @pl.when(i != 0)
    def _():
        gmax_ref[...] = jnp.maximum(gmax_ref[...], rmb)


def _tc_mid(accT, denT, b1c, W2, A2):
    return pl.pallas_call(
        _tc_mid_body,
        grid=(NP // NB,),
        in_specs=[
            pl.BlockSpec((256, NB), lambda i: (0, i)),
            pl.BlockSpec((4, NB), lambda i: (0, i)),
            pl.BlockSpec((256, 1), lambda i: (0, 0)),
            pl.BlockSpec((256, 128), lambda i: (0, 0)),
            pl.BlockSpec((128, 2), lambda i: (0, 0)),
        ],
        out_specs=[
            pl.BlockSpec((128, NB), lambda i: (0, i)),
            pl.BlockSpec((8, NB), lambda i: (0, i)),
            pl.BlockSpec((8, 128), lambda i: (0, 0)),
        ],
        out_shape=[
            jax.ShapeDtypeStruct((128, NP), F32),
            jax.ShapeDtypeStruct((8, NP), F32),
            jax.ShapeDtypeStruct((8, 128), F32),
        ],
    )(accT, denT, b1c, W2, A2)


def _tc_post_body(acc_ref, den_ref, b_ref, eye_ref, out_ref):
    acc = acc_ref[...]
    den = den_ref[...]
    h = acc / (lax.broadcast_in_dim(den, (128, NB), (0, 1)) + 1e-16)
    h = h + b_ref[...]
    h = jnp.where(h > 0, h, jnp.exp(h) - 1.0)
    out_ref[...] = lax.dot_general(h, eye_ref[...], (((0,), (0,)), ((), ())),
                                   preferred_element_type=F32)


def _tc_post(acc2T, den2, b2c, eye):
    return pl.pallas_call(
        _tc_post_body,
        grid=(NP // NB,),
        in_specs=[
            pl.BlockSpec((128, NB), lambda i: (0, i)),
            pl.BlockSpec((1, NB), lambda i: (0, i)),
            pl.BlockSpec((128, 1), lambda i: (0, 0)),
            pl.BlockSpec((128, 128), lambda i: (0, 0)),
        ],
        out_specs=pl.BlockSpec((NB, 128), lambda i: (i, 0)),
        out_shape=jax.ShapeDtypeStruct((NP, 128), F32),
    )(acc2T, den2, b2c, eye)


def _make_edge_kernel(heads, featc):
    nunits = featc // 4
    units_per_tec = nunits // 32
    dst_row = 4 if heads == 4 else 1
    chunks = EP // C
    if heads == 4:
        pha_range, pha_ca = EP // 4, 3072
    else:
        pha_range, pha_ca = EP // 16, 1344
    pha_chunks = pha_range // pha_ca

    @functools.partial(
        pl.kernel,
        out_type=(jax.ShapeDtypeStruct((featc * NP,), F32),
                  jax.ShapeDtypeStruct((8 * NP,), F32),
                  jax.ShapeDtypeStruct((heads * EP,), F32)),
        mesh=_mesh,
        compiler_params=_CP_SC,
        scratch_types=(
            [pltpu.VMEM((NP,), F32) for _ in range(4)]
            + [pltpu.VMEM((NP,), F32) for _ in range(4)]
            + [
                pltpu.VMEM((NP,), F32),
                pltpu.VMEM((C,), jnp.int32),
                pltpu.VMEM((C,), jnp.int32),
                pltpu.VMEM((C,), jnp.int32),
                pltpu.VMEM((C,), jnp.int32),
                pltpu.VMEM((C,), F32),
                pltpu.VMEM((C,), F32),
                pltpu.VMEM((128,), F32),
                pltpu.VMEM((128,), F32),
                pltpu.SemaphoreType.DMA,
                pltpu.SemaphoreType.DMA,
                pltpu.SemaphoreType.DMA,
                pltpu.SemaphoreType.DMA,
                pltpu.SemaphoreType.DMA,
                pltpu.SemaphoreType.DMA,
            ]
        ),
    )
    def edge_kernel(tabT, alphaT, gmaxrow, src, dst, accT_o, den_o, exS,
                    t0, t1, t2, t3, a0, a1, a2, a3,
                    accd, sv0, dv0, sv1, dv1, eb0, eb1, gm1, gm2,
                    ss0, sd0, ss1, sd1, se0, se1):
        tabs = (t0, t1, t2, t3)
        accs = (a0, a1, a2, a3)
        asr, ads = t0, t1
        cid = lax.axis_index("c")
        sid = lax.axis_index("s")
        wid = sid * 2 + cid
        zeros = jnp.zeros((16,), F32)

        if heads == 4:
            a_head = sid // 4
            a_sub = sid % 4
        else:
            a_head = sid * 0
            a_sub = sid
        a_base = a_head * EP + a_sub * pha_range
        pltpu.sync_copy(alphaT.at[pl.ds(a_head * NP, NP)], asr)
        pltpu.sync_copy(alphaT.at[pl.ds((dst_row + a_head) * NP, NP)], ads)
        pltpu.sync_copy(gmaxrow.at[pl.ds(a_head * 128, 128)], gm1)
        pltpu.sync_copy(gmaxrow.at[pl.ds((dst_row + a_head) * 128, 128)], gm2)
        b = gm1[pl.ds(0, 16)] + gm2[pl.ds(0, 16)]
        g = jnp.maximum(b, 0.2 * b)

        def _pha(i, carry):
            off = a_sub * pha_range + i * pha_ca
            pltpu.sync_copy(src.at[pl.ds(off, pha_ca)], sv0.at[pl.ds(0, pha_ca)])
            pltpu.sync_copy(dst.at[pl.ds(off, pha_ca)], dv0.at[pl.ds(0, pha_ca)])

            @plsc.parallel_loop(0, pha_ca, 16, unroll=4)
            def _body(o):
                s = sv0[pl.ds(o, 16)]
                d = dv0[pl.ds(o, 16)]
                e = plsc.load_gather(asr, [s]) + plsc.load_gather(ads, [d])
                e = jnp.maximum(e, 0.2 * e)
                eb0[pl.ds(o, 16)] = jnp.exp(e - g)

            pltpu.sync_copy(eb0.at[pl.ds(0, pha_ca)],
                            exS.at[pl.ds(a_base + i * pha_ca, pha_ca)])
            return carry

        lax.fori_loop(0, pha_chunks, _pha, 0)
        plsc.subcore_barrier()

        for t in range(units_per_tec):
            u = wid * units_per_tec + t
            head = (u // 16) if heads == 4 else (u * 0)
            for f in range(4):
                pltpu.sync_copy(tabT.at[pl.ds((u * 4 + f) * NP, NP)], tabs[f])

            @plsc.parallel_loop(0, NP, 16, unroll=8)
            def _zero(o):
                for f in range(4):
                    accs[f][pl.ds(o, 16)] = zeros
                accd[pl.ds(o, 16)] = zeros

            def _start(ci, svb, dvb, evb, sems):
                pltpu.async_copy(src.at[pl.ds(ci * C, C)], svb, sems[0])
                pltpu.async_copy(dst.at[pl.ds(ci * C, C)], dvb, sems[1])
                pltpu.async_copy(exS.at[pl.ds(head * EP + ci * C, C)],
                                 evb, sems[2])

            def _wait(svb, dvb, evb, sems):
                pltpu.make_async_copy(src.at[pl.ds(0, C)], svb, sems[0]).wait()
                pltpu.make_async_copy(dst.at[pl.ds(0, C)], dvb, sems[1]).wait()
                pltpu.make_async_copy(exS.at[pl.ds(0, C)], evb, sems[2]).wait()

            def _run(svb, dvb, evb):
                @plsc.parallel_loop(0, C, 16, unroll=4)
                def _body(o):
                    s = svb[pl.ds(o, 16)]
                    d = dvb[pl.ds(o, 16)]
                    ex = evb[pl.ds(o, 16)]
                    for f in range(4):
                        tv = plsc.load_gather(tabs[f], [s])
                        plsc.addupdate_scatter(accs[f], [d], tv * ex)
                    plsc.addupdate_scatter(accd, [d], ex)

            _start(0, sv0, dv0, eb0, (ss0, sd0, se0))

            def _pair(j, carry):
                ci = 2 * j
                _start(ci + 1, sv1, dv1, eb1, (ss1, sd1, se1))
                _wait(sv0, dv0, eb0, (ss0, sd0, se0))
                _run(sv0, dv0, eb0)
                _start(jnp.minimum(ci + 2, chunks - 1), sv0, dv0, eb0,
                       (ss0, sd0, se0))
                _wait(sv1, dv1, eb1, (ss1, sd1, se1))
                _run(sv1, dv1, eb1)
                return carry

            lax.fori_loop(0, chunks // 2, _pair, 0)
            _wait(sv0, dv0, eb0, (ss0, sd0, se0))
            for f in range(4):
                pltpu.sync_copy(accs[f], accT_o.at[pl.ds((u * 4 + f) * NP, NP)])
            is_aug = (u % 16 == 0) if heads == 4 else (u == 0)

            @pl.when(is_aug)
            def _():
                pltpu.sync_copy(accd, den_o.at[pl.ds(head * NP, NP)])

    return edge_kernel


_edge_l1 = _make_edge_kernel(4, 256)
_edge_l2 = _make_edge_kernel(1, 128)


def _branch(x, edge_index, p1, p2):
    W1, as1, ad1, b1 = p1
    W2, as2, ad2, b2 = p2

    loop = jnp.arange(N, dtype=edge_index.dtype)
    src = jnp.concatenate([edge_index[0], loop])
    dst = jnp.concatenate([edge_index[1], loop])
    pad = jnp.full((EP - E - N,), NP - 1, dtype=src.dtype)
    src = jnp.concatenate([src, pad])
    dst = jnp.concatenate([dst, pad])

    xp = jnp.pad(x, ((0, NP - N), (0, 0)))

    eye4 = jnp.eye(4, dtype=F32)
    A1s = jnp.einsum("hc,hk->hck", as1, eye4).reshape(256, 4)
    A1d = jnp.einsum("hc,hk->hck", ad1, eye4).reshape(256, 4)
    A1 = jnp.concatenate([A1s, A1d], axis=1)
    A2 = jnp.stack([as2[0], ad2[0]], axis=1)

    h1T, alphaT, gmaxrow = _tc_pre(xp, W1, A1, 256)
    accT, denT, _ex1 = _edge_l1(h1T.reshape(-1), alphaT.reshape(-1),
                                gmaxrow.reshape(-1), src, dst)
    zT, alphaT2, gmax2row = _tc_mid(accT.reshape(256, NP),
                                    denT.reshape(8, NP)[:4],
                                    b1[:, None], W2, A2)
    acc2T, den2, _ex2 = _edge_l2(zT.reshape(-1), alphaT2.reshape(-1),
                                 gmax2row.reshape(-1), src, dst)
    outp = _tc_post(acc2T.reshape(128, NP), den2.reshape(8, NP)[:1],
                    b2[:, None], jnp.eye(128, dtype=F32))
    return outp[:N]


def kernel(x0, x1, edge_index0, edge_index1, W1_0, as1_0, ad1_0, b1_0, W2_0, as2_0, ad2_0, b2_0, W1_1, as1_1, ad1_1, b1_1, W2_1, as2_1, ad2_1, b2_1):
    out0 = _branch(x0, edge_index0, (W1_0, as1_0, ad1_0, b1_0), (W2_0, as2_0, ad2_0, b2_0))
    out1 = _branch(x1, edge_index1, (W1_1, as1_1, ad1_1, b1_1), (W2_1, as2_1, ad2_1, b2_1))
    return jnp.concatenate([out0, out1], axis=0)


# speedup vs baseline: 1.1108x; 1.0873x over previous
import functools

import jax
import jax.numpy as jnp
from jax import lax
from jax.experimental import pallas as pl
from jax.experimental.pallas import tpu as pltpu
from jax.experimental.pallas import tpu_sc as plsc

N = 10000
NP = 10240
E = 160000
EP = 172032
C = 4096
NB = 1024
F32 = jnp.float32

_mesh = plsc.VectorSubcoreMesh(core_axis_name="c", subcore_axis_name="s")
_CP_SC = pltpu.CompilerParams(needs_layout_passes=False)


def _tc_pre_body(x_ref, w_ref, a_ref, hT_ref, al_ref, gmax_ref):
    hT = lax.dot_general(w_ref[...], x_ref[...], (((0,), (1,)), ((), ())),
                         preferred_element_type=F32)
    hT_ref[...] = hT
    al = lax.dot_general(a_ref[...], hT, (((0,), (0,)), ((), ())),
                         preferred_element_type=F32)
    al_ref[...] = al
    rm = jnp.max(al, axis=1, keepdims=True)
    rmb = lax.broadcast_in_dim(rm, (8, 128), (0, 1))

    @pl.when(pl.program_id(0) == 0)
    def _():
        gmax_ref[...] = rmb

    @pl.when(pl.program_id(0) != 0)
    def _():
        gmax_ref[...] = jnp.maximum(gmax_ref[...], rmb)


def _tc_pre(xp, W, A, dh):
    din = xp.shape[1]
    return pl.pallas_call(
        _tc_pre_body,
        grid=(NP // NB,),
        in_specs=[
            pl.BlockSpec((NB, din), lambda i: (i, 0)),
            pl.BlockSpec((din, dh), lambda i: (0, 0)),
            pl.BlockSpec((dh, 8), lambda i: (0, 0)),
        ],
        out_specs=[
            pl.BlockSpec((dh, NB), lambda i: (0, i)),
            pl.BlockSpec((8, NB), lambda i: (0, i)),
            pl.BlockSpec((8, 128), lambda i: (0, 0)),
        ],
        out_shape=[
            jax.ShapeDtypeStruct((dh, NP), F32),
            jax.ShapeDtypeStruct((8, NP), F32),
            jax.ShapeDtypeStruct((8, 128), F32),
        ],
    )(xp, W, A)


def _tc_mid_body(acc_ref, den_ref, b_ref, w_ref, a_ref,
                 zT_ref, al_ref, gmax_ref):
    i = pl.program_id(0)
    acc = acc_ref[...]
    den = den_ref[...]
    col = lax.broadcasted_iota(jnp.int32, (1, NB), 1) + i * NB
    valid = col < N
    acc = jnp.where(lax.broadcast_in_dim(valid, (256, NB), (0, 1)), acc, 0.0)
    den = jnp.where(lax.broadcast_in_dim(valid, (4, NB), (0, 1)), den, 1.0)
    acc3 = acc.reshape(4, 64, NB)
    den3 = lax.broadcast_in_dim(den, (4, 64, NB), (0, 2))
    h = acc3 / (den3 + 1e-16) + b_ref[...].reshape(4, 64, 1)
    h = h.reshape(256, NB)
    h = jnp.where(h > 0, h, jnp.exp(h) - 1.0)
    z = lax.dot_general(w_ref[...], h, (((0,), (0,)), ((), ())),
                        preferred_element_type=F32)
    zT_ref[...] = z
    al2 = lax.dot_general(a_ref[...], z, (((0,), (0,)), ((), ())),
                          preferred_element_type=F32)
    al2p = jnp.concatenate([al2, jnp.full((6, NB), -1e30, F32)], axis=0)
    al_ref[...] = al2p
    rm = jnp.max(al2p, axis=1, keepdims=True)
    rmb = lax.broadcast_in_dim(rm, (8, 128), (0, 1))

    @pl.when(i == 0)
    def _():
        gmax_ref[...] = rmb

    @pl.when(i != 0)
    def _():
        gmax_ref[...] = jnp.maximum(gmax_ref[...], rmb)


def _tc_mid(accT, denT, b1c, W2, A2):
    return pl.pallas_call(
        _tc_mid_body,
        grid=(NP // NB,),
        in_specs=[
            pl.BlockSpec((256, NB), lambda i: (0, i)),
            pl.BlockSpec((4, NB), lambda i: (0, i)),
            pl.BlockSpec((256, 1), lambda i: (0, 0)),
            pl.BlockSpec((256, 128), lambda i: (0, 0)),
            pl.BlockSpec((128, 2), lambda i: (0, 0)),
        ],
        out_specs=[
            pl.BlockSpec((128, NB), lambda i: (0, i)),
            pl.BlockSpec((8, NB), lambda i: (0, i)),
            pl.BlockSpec((8, 128), lambda i: (0, 0)),
        ],
        out_shape=[
            jax.ShapeDtypeStruct((128, NP), F32),
            jax.ShapeDtypeStruct((8, NP), F32),
            jax.ShapeDtypeStruct((8, 128), F32),
        ],
    )(accT, denT, b1c, W2, A2)


def _tc_post_body(acc_ref, den_ref, b_ref, eye_ref, out_ref):
    acc = acc_ref[...]
    den = den_ref[...]
    h = acc / (lax.broadcast_in_dim(den, (128, NB), (0, 1)) + 1e-16)
    h = h + b_ref[...]
    h = jnp.where(h > 0, h, jnp.exp(h) - 1.0)
    out_ref[...] = lax.dot_general(h, eye_ref[...], (((0,), (0,)), ((), ())),
                                   preferred_element_type=F32)


def _tc_post(acc2T, den2, b2c, eye):
    return pl.pallas_call(
        _tc_post_body,
        grid=(NP // NB,),
        in_specs=[
            pl.BlockSpec((128, NB), lambda i: (0, i)),
            pl.BlockSpec((1, NB), lambda i: (0, i)),
            pl.BlockSpec((128, 1), lambda i: (0, 0)),
            pl.BlockSpec((128, 128), lambda i: (0, 0)),
        ],
        out_specs=pl.BlockSpec((NB, 128), lambda i: (i, 0)),
        out_shape=jax.ShapeDtypeStruct((NP, 128), F32),
    )(acc2T, den2, b2c, eye)


def _make_edge_kernel(heads, featc):
    nunits = featc // 4
    units_per_tec = nunits // 32
    dst_row = 4 if heads == 4 else 1
    chunks = EP // C

    @functools.partial(
        pl.kernel,
        out_type=(jax.ShapeDtypeStruct((featc * NP,), F32),
                  jax.ShapeDtypeStruct((8 * NP,), F32)),
        mesh=_mesh,
        compiler_params=_CP_SC,
        scratch_types=(
            [pltpu.VMEM((NP,), F32) for _ in range(4)]
            + [pltpu.VMEM((NP,), F32) for _ in range(4)]
            + [
                pltpu.VMEM((NP,), F32),
                pltpu.VMEM((NP,), F32),
                pltpu.VMEM((NP,), F32),
                pltpu.VMEM((C,), jnp.int32),
                pltpu.VMEM((C,), jnp.int32),
                pltpu.VMEM((C,), jnp.int32),
                pltpu.VMEM((C,), jnp.int32),
                pltpu.VMEM((128,), F32),
                pltpu.VMEM((128,), F32),
                pltpu.SemaphoreType.DMA,
                pltpu.SemaphoreType.DMA,
                pltpu.SemaphoreType.DMA,
                pltpu.SemaphoreType.DMA,
            ]
        ),
    )
    def edge_kernel(tabT, alphaT, gmaxrow, src, dst, accT_o, den_o,
                    t0, t1, t2, t3, a0, a1, a2, a3,
                    asr, ads, accd, sv0, dv0, sv1, dv1, gm1, gm2,
                    ss0, sd0, ss1, sd1):
        tabs = (t0, t1, t2, t3)
        accs = (a0, a1, a2, a3)
        cid = lax.axis_index("c")
        sid = lax.axis_index("s")
        wid = sid * 2 + cid
        zeros = jnp.zeros((16,), F32)
        for t in range(units_per_tec):
            u = wid * units_per_tec + t
            head = (u // 16) if heads == 4 else (u * 0)
            for f in range(4):
                pltpu.sync_copy(tabT.at[pl.ds((u * 4 + f) * NP, NP)], tabs[f])
            pltpu.sync_copy(alphaT.at[pl.ds(head * NP, NP)], asr)
            pltpu.sync_copy(alphaT.at[pl.ds((dst_row + head) * NP, NP)], ads)
            pltpu.sync_copy(gmaxrow.at[pl.ds(head * 128, 128)], gm1)
            pltpu.sync_copy(gmaxrow.at[pl.ds((dst_row + head) * 128, 128)], gm2)
            b = gm1[pl.ds(0, 16)] + gm2[pl.ds(0, 16)]
            g = jnp.maximum(b, 0.2 * b)

            @plsc.parallel_loop(0, NP, 16, unroll=8)
            def _zero(o):
                for f in range(4):
                    accs[f][pl.ds(o, 16)] = zeros
                accd[pl.ds(o, 16)] = zeros

            def _start(ci, svb, dvb, sems):
                pltpu.async_copy(src.at[pl.ds(ci * C, C)], svb, sems[0])
                pltpu.async_copy(dst.at[pl.ds(ci * C, C)], dvb, sems[1])

            def _wait(svb, dvb, sems):
                pltpu.make_async_copy(src.at[pl.ds(0, C)], svb, sems[0]).wait()
                pltpu.make_async_copy(dst.at[pl.ds(0, C)], dvb, sems[1]).wait()

            def _run(svb, dvb, with_den):
                @plsc.parallel_loop(0, C, 16, unroll=4)
                def _body(o):
                    s = svb[pl.ds(o, 16)]
                    d = dvb[pl.ds(o, 16)]
                    e = plsc.load_gather(asr, [s]) + plsc.load_gather(ads, [d])
                    e = jnp.maximum(e, 0.2 * e)
                    ex = jnp.exp(e - g)
                    for f in range(4):
                        tv = plsc.load_gather(tabs[f], [s])
                        plsc.addupdate_scatter(accs[f], [d], tv * ex)
                    if with_den:
                        plsc.addupdate_scatter(accd, [d], ex)

            def _edge_sweep(with_den):
                _start(0, sv0, dv0, (ss0, sd0))

                def _pair(j, carry):
                    ci = 2 * j
                    _start(ci + 1, sv1, dv1, (ss1, sd1))
                    _wait(sv0, dv0, (ss0, sd0))
                    _run(sv0, dv0, with_den)
                    _start(jnp.minimum(ci + 2, chunks - 1), sv0, dv0,
                           (ss0, sd0))
                    _wait(sv1, dv1, (ss1, sd1))
                    _run(sv1, dv1, with_den)
                    return carry

                lax.fori_loop(0, chunks // 2, _pair, 0)
                _wait(sv0, dv0, (ss0, sd0))

            is_aug = (u % 16 == 0) if heads == 4 else (u == 0)

            @pl.when(is_aug)
            def _():
                _edge_sweep(True)

            @pl.when(jnp.logical_not(is_aug))
            def _():
                _edge_sweep(False)

            for f in range(4):
                pltpu.sync_copy(accs[f], accT_o.at[pl.ds((u * 4 + f) * NP, NP)])

            @pl.when(is_aug)
            def _():
                pltpu.sync_copy(accd, den_o.at[pl.ds(head * NP, NP)])

    return edge_kernel


_edge_l1 = _make_edge_kernel(4, 256)
_edge_l2 = _make_edge_kernel(1, 128)


def _branch(x, edge_index, p1, p2):
    W1, as1, ad1, b1 = p1
    W2, as2, ad2, b2 = p2

    loop = jnp.arange(N, dtype=edge_index.dtype)
    src = jnp.concatenate([edge_index[0], loop])
    dst = jnp.concatenate([edge_index[1], loop])
    pad = jnp.full((EP - E - N,), NP - 1, dtype=src.dtype)
    src = jnp.concatenate([src, pad])
    dst = jnp.concatenate([dst, pad])

    xp = jnp.pad(x, ((0, NP - N), (0, 0)))

    eye4 = jnp.eye(4, dtype=F32)
    A1s = jnp.einsum("hc,hk->hck", as1, eye4).reshape(256, 4)
    A1d = jnp.einsum("hc,hk->hck", ad1, eye4).reshape(256, 4)
    A1 = jnp.concatenate([A1s, A1d], axis=1)
    A2 = jnp.stack([as2[0], ad2[0]], axis=1)

    h1T, alphaT, gmaxrow = _tc_pre(xp, W1, A1, 256)
    accT, denT = _edge_l1(h1T.reshape(-1), alphaT.reshape(-1),
                          gmaxrow.reshape(-1), src, dst)
    zT, alphaT2, gmax2row = _tc_mid(accT.reshape(256, NP),
                                    denT.reshape(8, NP)[:4],
                                    b1[:, None], W2, A2)
    acc2T, den2 = _edge_l2(zT.reshape(-1), alphaT2.reshape(-1),
                           gmax2row.reshape(-1), src, dst)
    outp = _tc_post(acc2T.reshape(128, NP), den2.reshape(8, NP)[:1],
                    b2[:, None], jnp.eye(128, dtype=F32))
    return outp[:N]


def kernel(x0, x1, edge_index0, edge_index1, W1_0, as1_0, ad1_0, b1_0, W2_0, as2_0, ad2_0, b2_0, W1_1, as1_1, ad1_1, b1_1, W2_1, as2_1, ad2_1, b2_1):
    out0 = _branch(x0, edge_index0, (W1_0, as1_0, ad1_0, b1_0), (W2_0, as2_0, ad2_0, b2_0))
    out1 = _branch(x1, edge_index1, (W1_1, as1_1, ad1_1, b1_1), (W2_1, as2_1, ad2_1, b2_1))
    return jnp.concatenate([out0, out1], axis=0)


# speedup vs baseline: 1.1412x; 1.0274x over previous
import functools

import jax
import jax.numpy as jnp
from jax import lax
from jax.experimental import pallas as pl
from jax.experimental.pallas import tpu as pltpu
from jax.experimental.pallas import tpu_sc as plsc

N = 10000
NP = 10240
E = 160000
EP = 172032
C = 4096
NB = 1024
F32 = jnp.float32

_mesh = plsc.VectorSubcoreMesh(core_axis_name="c", subcore_axis_name="s")
_CP_SC = pltpu.CompilerParams(needs_layout_passes=False)


def _tc_pre_body(x_ref, w_ref, a_ref, hT_ref, al_ref, gmax_ref):
    hT = lax.dot_general(w_ref[...], x_ref[...], (((0,), (1,)), ((), ())),
                         preferred_element_type=F32)
    hT_ref[...] = hT
    al = lax.dot_general(a_ref[...], hT, (((0,), (0,)), ((), ())),
                         preferred_element_type=F32)
    al_ref[...] = al
    rm = jnp.max(al, axis=1, keepdims=True)
    rmb = lax.broadcast_in_dim(rm, (8, 128), (0, 1))

    @pl.when(pl.program_id(0) == 0)
    def _():
        gmax_ref[...] = rmb

    @pl.when(pl.program_id(0) != 0)
    def _():
        gmax_ref[...] = jnp.maximum(gmax_ref[...], rmb)


def _tc_pre(xp, W, A, dh):
    din = xp.shape[1]
    return pl.pallas_call(
        _tc_pre_body,
        grid=(NP // NB,),
        in_specs=[
            pl.BlockSpec((NB, din), lambda i: (i, 0)),
            pl.BlockSpec((din, dh), lambda i: (0, 0)),
            pl.BlockSpec((dh, 8), lambda i: (0, 0)),
        ],
        out_specs=[
            pl.BlockSpec((dh, NB), lambda i: (0, i)),
            pl.BlockSpec((8, NB), lambda i: (0, i)),
            pl.BlockSpec((8, 128), lambda i: (0, 0)),
        ],
        out_shape=[
            jax.ShapeDtypeStruct((dh, NP), F32),
            jax.ShapeDtypeStruct((8, NP), F32),
            jax.ShapeDtypeStruct((8, 128), F32),
        ],
    )(xp, W, A)


def _tc_mid_body(acc_ref, den_ref, b_ref, w_ref, a_ref,
                 zT_ref, al_ref, gmax_ref):
    i = pl.program_id(0)
    acc = acc_ref[...]
    den = den_ref[...]
    col = lax.broadcasted_iota(jnp.int32, (1, NB), 1) + i * NB
    valid = col < N
    acc = jnp.where(lax.broadcast_in_dim(valid, (256, NB), (0, 1)), acc, 0.0)
    den = jnp.where(lax.broadcast_in_dim(valid, (4, NB), (0, 1)), den, 1.0)
    acc3 = acc.reshape(4, 64, NB)
    den3 = lax.broadcast_in_dim(den, (4, 64, NB), (0, 2))
    h = acc3 / (den3 + 1e-16) + b_ref[...].reshape(4, 64, 1)
    h = h.reshape(256, NB)
    h = jnp.where(h > 0, h, jnp.exp(h) - 1.0)
    z = lax.dot_general(w_ref[...], h, (((0,), (0,)), ((), ())),
                        preferred_element_type=F32)
    zT_ref[...] = z
    al2 = lax.dot_general(a_ref[...], z, (((0,), (0,)), ((), ())),
                          preferred_element_type=F32)
    al2p = jnp.concatenate([al2, jnp.full((6, NB), -1e30, F32)], axis=0)
    al_ref[...] = al2p
    rm = jnp.max(al2p, axis=1, keepdims=True)
    rmb = lax.broadcast_in_dim(rm, (8, 128), (0, 1))

    @pl.when(i == 0)
    def _():
        gmax_ref[...] = rmb

    @pl.when(i != 0)
    def _():
        gmax_ref[...] = jnp.maximum(gmax_ref[...], rmb)


def _tc_mid(accT, denT, b1c, W2, A2):
    return pl.pallas_call(
        _tc_mid_body,
        grid=(NP // NB,),
        in_specs=[
            pl.BlockSpec((256, NB), lambda i: (0, i)),
            pl.BlockSpec((4, NB), lambda i: (0, i)),
            pl.BlockSpec((256, 1), lambda i: (0, 0)),
            pl.BlockSpec((256, 128), lambda i: (0, 0)),
            pl.BlockSpec((128, 2), lambda i: (0, 0)),
        ],
        out_specs=[
            pl.BlockSpec((128, NB), lambda i: (0, i)),
            pl.BlockSpec((8, NB), lambda i: (0, i)),
            pl.BlockSpec((8, 128), lambda i: (0, 0)),
        ],
        out_shape=[
            jax.ShapeDtypeStruct((128, NP), F32),
            jax.ShapeDtypeStruct((8, NP), F32),
            jax.ShapeDtypeStruct((8, 128), F32),
        ],
    )(accT, denT, b1c, W2, A2)


def _tc_post_body(acc_ref, den_ref, b_ref, eye_ref, out_ref):
    acc = acc_ref[...]
    den = den_ref[...]
    h = acc / (lax.broadcast_in_dim(den, (128, NB), (0, 1)) + 1e-16)
    h = h + b_ref[...]
    h = jnp.where(h > 0, h, jnp.exp(h) - 1.0)
    out_ref[...] = lax.dot_general(h, eye_ref[...], (((0,), (0,)), ((), ())),
                                   preferred_element_type=F32)


def _tc_post(acc2T, den2, b2c, eye):
    return pl.pallas_call(
        _tc_post_body,
        grid=(NP // NB,),
        in_specs=[
            pl.BlockSpec((128, NB), lambda i: (0, i)),
            pl.BlockSpec((1, NB), lambda i: (0, i)),
            pl.BlockSpec((128, 1), lambda i: (0, 0)),
            pl.BlockSpec((128, 128), lambda i: (0, 0)),
        ],
        out_specs=pl.BlockSpec((NB, 128), lambda i: (i, 0)),
        out_shape=jax.ShapeDtypeStruct((NP, 128), F32),
    )(acc2T, den2, b2c, eye)


def _make_edge_kernel(heads, featc):
    nunits = featc // 4
    units_per_tec = nunits // 32
    dst_row = 4 if heads == 4 else 1
    chunks = EP // C

    @functools.partial(
        pl.kernel,
        out_type=(jax.ShapeDtypeStruct((featc * NP,), F32),
                  jax.ShapeDtypeStruct((8 * NP,), F32)),
        mesh=_mesh,
        compiler_params=_CP_SC,
        scratch_types=(
            [pltpu.VMEM((NP,), F32) for _ in range(4)]
            + [pltpu.VMEM((NP,), F32) for _ in range(4)]
            + [
                pltpu.VMEM((NP,), F32),
                pltpu.VMEM((NP,), F32),
                pltpu.VMEM((NP,), F32),
                pltpu.VMEM((C,), jnp.int32),
                pltpu.VMEM((C,), jnp.int32),
                pltpu.VMEM((C,), jnp.int32),
                pltpu.VMEM((C,), jnp.int32),
                pltpu.VMEM((128,), F32),
                pltpu.VMEM((128,), F32),
                pltpu.SemaphoreType.DMA,
                pltpu.SemaphoreType.DMA,
                pltpu.SemaphoreType.DMA,
                pltpu.SemaphoreType.DMA,
            ]
        ),
    )
    def edge_kernel(tabT, alphaT, gmaxrow, src, dst, accT_o, den_o,
                    t0, t1, t2, t3, a0, a1, a2, a3,
                    asr, ads, accd, sv0, dv0, sv1, dv1, gm1, gm2,
                    ss0, sd0, ss1, sd1):
        tabs = (t0, t1, t2, t3)
        accs = (a0, a1, a2, a3)
        cid = lax.axis_index("c")
        sid = lax.axis_index("s")
        wid = sid * 2 + cid
        zeros = jnp.zeros((16,), F32)
        for t in range(units_per_tec):
            u = wid * units_per_tec + t
            head = (u // 16) if heads == 4 else (u * 0)
            setup = []
            for f in range(4):
                setup.append(pltpu.async_copy(
                    tabT.at[pl.ds((u * 4 + f) * NP, NP)], tabs[f], ss0))
            setup.append(pltpu.async_copy(
                alphaT.at[pl.ds(head * NP, NP)], asr, sd0))
            setup.append(pltpu.async_copy(
                alphaT.at[pl.ds((dst_row + head) * NP, NP)], ads, ss1))
            setup.append(pltpu.async_copy(
                gmaxrow.at[pl.ds(head * 128, 128)], gm1, sd1))
            setup.append(pltpu.async_copy(
                gmaxrow.at[pl.ds((dst_row + head) * 128, 128)], gm2, sd1))

            @plsc.parallel_loop(0, NP, 16, unroll=8)
            def _zero(o):
                for f in range(4):
                    accs[f][pl.ds(o, 16)] = zeros
                accd[pl.ds(o, 16)] = zeros

            for d in setup:
                d.wait()
            b = gm1[pl.ds(0, 16)] + gm2[pl.ds(0, 16)]
            g = jnp.maximum(b, 0.2 * b)

            def _start(ci, svb, dvb, sems):
                pltpu.async_copy(src.at[pl.ds(ci * C, C)], svb, sems[0])
                pltpu.async_copy(dst.at[pl.ds(ci * C, C)], dvb, sems[1])

            def _wait(svb, dvb, sems):
                pltpu.make_async_copy(src.at[pl.ds(0, C)], svb, sems[0]).wait()
                pltpu.make_async_copy(dst.at[pl.ds(0, C)], dvb, sems[1]).wait()

            def _run(svb, dvb, with_den):
                @plsc.parallel_loop(0, C, 16, unroll=4)
                def _body(o):
                    s = svb[pl.ds(o, 16)]
                    d = dvb[pl.ds(o, 16)]
                    e = plsc.load_gather(asr, [s]) + plsc.load_gather(ads, [d])
                    e = jnp.maximum(e, 0.2 * e)
                    ex = jnp.exp(e - g)
                    for f in range(4):
                        tv = plsc.load_gather(tabs[f], [s])
                        plsc.addupdate_scatter(accs[f], [d], tv * ex)
                    if with_den:
                        plsc.addupdate_scatter(accd, [d], ex)

            def _edge_sweep(with_den):
                _start(0, sv0, dv0, (ss0, sd0))

                def _pair(j, carry):
                    ci = 2 * j
                    _start(ci + 1, sv1, dv1, (ss1, sd1))
                    _wait(sv0, dv0, (ss0, sd0))
                    _run(sv0, dv0, with_den)
                    _start(jnp.minimum(ci + 2, chunks - 1), sv0, dv0,
                           (ss0, sd0))
                    _wait(sv1, dv1, (ss1, sd1))
                    _run(sv1, dv1, with_den)
                    return carry

                lax.fori_loop(0, chunks // 2, _pair, 0)
                _wait(sv0, dv0, (ss0, sd0))

            if heads == 4:
                is_aug = ((u == 2) | (u == 16) | (u == 34) | (u == 48))
            else:
                is_aug = (u == 0)

            @pl.when(is_aug)
            def _():
                _edge_sweep(True)

            @pl.when(jnp.logical_not(is_aug))
            def _():
                _edge_sweep(False)

            for f in range(4):
                pltpu.sync_copy(accs[f], accT_o.at[pl.ds((u * 4 + f) * NP, NP)])

            @pl.when(is_aug)
            def _():
                pltpu.sync_copy(accd, den_o.at[pl.ds(head * NP, NP)])

    return edge_kernel


_edge_l1 = _make_edge_kernel(4, 256)
_edge_l2 = _make_edge_kernel(1, 128)


def _branch(x, edge_index, p1, p2):
    W1, as1, ad1, b1 = p1
    W2, as2, ad2, b2 = p2

    loop = jnp.arange(N, dtype=edge_index.dtype)
    src = jnp.concatenate([edge_index[0], loop])
    dst = jnp.concatenate([edge_index[1], loop])
    pad = jnp.full((EP - E - N,), NP - 1, dtype=src.dtype)
    src = jnp.concatenate([src, pad])
    dst = jnp.concatenate([dst, pad])

    xp = jnp.pad(x, ((0, NP - N), (0, 0)))

    eye4 = jnp.eye(4, dtype=F32)
    A1s = jnp.einsum("hc,hk->hck", as1, eye4).reshape(256, 4)
    A1d = jnp.einsum("hc,hk->hck", ad1, eye4).reshape(256, 4)
    A1 = jnp.concatenate([A1s, A1d], axis=1)
    A2 = jnp.stack([as2[0], ad2[0]], axis=1)

    h1T, alphaT, gmaxrow = _tc_pre(xp, W1, A1, 256)
    accT, denT = _edge_l1(h1T.reshape(-1), alphaT.reshape(-1),
                          gmaxrow.reshape(-1), src, dst)
    zT, alphaT2, gmax2row = _tc_mid(accT.reshape(256, NP),
                                    denT.reshape(8, NP)[:4],
                                    b1[:, None], W2, A2)
    acc2T, den2 = _edge_l2(zT.reshape(-1), alphaT2.reshape(-1),
                           gmax2row.reshape(-1), src, dst)
    outp = _tc_post(acc2T.reshape(128, NP), den2.reshape(8, NP)[:1],
                    b2[:, None], jnp.eye(128, dtype=F32))
    return outp[:N]


def kernel(x0, x1, edge_index0, edge_index1, W1_0, as1_0, ad1_0, b1_0, W2_0, as2_0, ad2_0, b2_0, W1_1, as1_1, ad1_1, b1_1, W2_1, as2_1, ad2_1, b2_1):
    out0 = _branch(x0, edge_index0, (W1_0, as1_0, ad1_0, b1_0), (W2_0, as2_0, ad2_0, b2_0))
    out1 = _branch(x1, edge_index1, (W1_1, as1_1, ad1_1, b1_1), (W2_1, as2_1, ad2_1, b2_1))
    return jnp.concatenate([out0, out1], axis=0)
